# baseline (speedup 1.0000x reference)
import jax
import jax.numpy as jnp
from jax import lax
from jax.experimental import pallas as pl
from jax.experimental.pallas import tpu as pltpu
from jax.experimental.pallas import tpu_sc as plsc

_N = 10000
_E = 320000
_D = 128
_NC = 2
_NS = 16
_NW = _NC * _NS
_CH = 64
_NCHUNK = 5120
_EPAD = _NCHUNK * _CH
_EPW = _NCHUNK // _NW
_B = 8
_NBLK = _EPW // _B
_NBUF = 4
_NP = 10240
_RPT = _NP // _NS
_R = 640
_NB = _NP // _R


def _build_agg(with_cnt):
    mesh = plsc.VectorSubcoreMesh(core_axis_name="c", subcore_axis_name="s")
    out_type = [jax.ShapeDtypeStruct((_NC * _NP, _D), jnp.float32)]
    scratch = [
        pltpu.VMEM((2, _B, _CH), jnp.int32),
        pltpu.VMEM((2, _B, _CH), jnp.int32),
        pltpu.VMEM((_NBUF, _CH, _D), jnp.float32),
        pltpu.VMEM_SHARED((_NP, _D), jnp.float32),
    ] + [pltpu.SemaphoreType.DMA] * (2 * _NBUF + 1)
    if with_cnt:
        out_type.append(jax.ShapeDtypeStruct((_NW * _NP,), jnp.float32))
        scratch.append(pltpu.VMEM((_NP,), jnp.float32))

    def body(*refs):
        if with_cnt:
            (src_h, dst_h, tab_h, zrow_h,
             agg_h, hist_h,
             src_a, dst_a, rows, acc, *sems_all, hist) = refs
        else:
            (src_h, dst_h, tab_h, zrow_h,
             agg_h,
             src_a, dst_a, rows, acc, *sems_all) = refs
        gsems = sems_all[:_NBUF]
        ssems = sems_all[_NBUF:2 * _NBUF]
        sem_i = sems_all[2 * _NBUF]
        c = lax.axis_index("c")
        s = lax.axis_index("s")
        wid = s * _NC + c
        base = s * _RPT
        brow0 = wid * _EPW

        def idx_block(kb, slot, do):
            r = brow0 + kb * _B
            a = pltpu.make_async_copy(src_h.at[pl.ds(r, _B)], src_a.at[slot],
                                      sem_i)
            b = pltpu.make_async_copy(dst_h.at[pl.ds(r, _B)], dst_a.at[slot],
                                      sem_i)
            if do == "start":
                a.start()
                b.start()
            else:
                a.wait()
                b.wait()

        pltpu.sync_copy(src_h.at[pl.ds(brow0, _B)], src_a.at[0])
        pltpu.sync_copy(dst_h.at[pl.ds(brow0, _B)], dst_a.at[0])
        pltpu.sync_copy(zrow_h.at[pl.ds(base, _RPT)], acc.at[pl.ds(base, _RPT)])
        if with_cnt:
            def zstep(i, carry):
                hist[pl.ds(i * 16, 16)] = jnp.zeros((16,), jnp.float32)
                return carry
            lax.fori_loop(0, _NP // 16, zstep, 0)
        plsc.subcore_barrier()

        pltpu.async_copy(tab_h.at[src_a.at[0, 0]], rows.at[0], gsems[0])
        pltpu.async_copy(tab_h.at[src_a.at[0, 1]], rows.at[1], gsems[1])
        idx_block(1, 1, "start")

        def step(kb, carry):
            cur = lax.rem(kb, 2)
            nxt = 1 - cur

            @pl.when(kb + 1 < _NBLK)
            def _():
                idx_block(kb + 1, nxt, "wait")

            for j in range(_B):
                b = j % _NBUF
                bb = (j + 2) % _NBUF
                pltpu.make_async_copy(tab_h.at[src_a.at[cur, j]],
                                      rows.at[b], gsems[b]).wait()
                if j < _B - 2:
                    pltpu.async_copy(rows.at[b], acc.at[dst_a.at[cur, j]],
                                     ssems[b], add=True)
                else:
                    pltpu.sync_copy(rows.at[b], acc.at[dst_a.at[cur, j]],
                                    add=True)
                if with_cnt:
                    for i in range(_CH // 16):
                        v = dst_a[cur, j, pl.ds(i * 16, 16)]
                        plsc.addupdate_scatter(
                            hist, [v], jnp.ones((16,), jnp.float32))

                def drain_bb():
                    pltpu.make_async_copy(rows.at[bb],
                                          acc.at[dst_a.at[0, 0]],
                                          ssems[bb]).wait()
                if j < _B - 2:
                    if j >= 2:
                        drain_bb()
                    pltpu.async_copy(tab_h.at[src_a.at[cur, j + 2]],
                                     rows.at[bb], gsems[bb])
                else:
                    @pl.when(kb + 1 < _NBLK)
                    def _():
                        drain_bb()
                        pltpu.async_copy(tab_h.at[src_a.at[nxt, j + 2 - _B]],
                                         rows.at[bb], gsems[bb])

            @pl.when(kb + 2 < _NBLK)
            def _():
                idx_block(kb + 2, cur, "start")
            return carry

        lax.fori_loop(0, _NBLK, step, 0)
        pltpu.make_async_copy(rows.at[0], acc.at[dst_a.at[0, 0]],
                              ssems[0]).wait()
        pltpu.make_async_copy(rows.at[1], acc.at[dst_a.at[0, 0]],
                              ssems[1]).wait()
        plsc.subcore_barrier()
        orow = c * _NP + base
        pltpu.sync_copy(acc.at[pl.ds(base, _RPT)], agg_h.at[pl.ds(orow, _RPT)])
        if with_cnt:
            pltpu.sync_copy(hist, hist_h.at[pl.ds(wid * _NP, _NP)])

    return pl.kernel(body, out_type=tuple(out_type), mesh=mesh,
                     scratch_types=tuple(scratch),
                     compiler_params=pltpu.CompilerParams(
                         needs_layout_passes=not with_cnt))


_AGG_CNT = _build_agg(True)
_AGG = _build_agg(False)

_CONTRACT_T = (((1,), (1,)), ((), ()))


def _tc_layer1(aggP, hist, x, W1_l, b1, W1_r, W2_r):
    def body(a0, a1, h_r, x_r, wl, br, wr, w2r, h1_o, h1r_o, cnt_o):
        cnt = jnp.maximum(jnp.sum(h_r[...], axis=1, keepdims=True), 1.0)
        mean = (a0[...] + a1[...]) / cnt
        t = lax.dot_general(mean, wl[...], _CONTRACT_T,
                            preferred_element_type=jnp.float32)
        u = lax.dot_general(x_r[...], wr[...], _CONTRACT_T,
                            preferred_element_type=jnp.float32)
        h1 = jnp.maximum(t + br[...] + u, 0.0)
        h1_o[...] = h1
        h1r_o[...] = lax.dot_general(h1, w2r[...], _CONTRACT_T,
                                     preferred_element_type=jnp.float32)
        cnt_o[...] = cnt

    grid = (_NB,)
    return pl.pallas_call(
        body,
        grid=grid,
        in_specs=[
            pl.BlockSpec((_R, _D), lambda i: (i, 0)),
            pl.BlockSpec((_R, _D), lambda i: (i + _NB, 0)),
            pl.BlockSpec((_R, _NW), lambda i: (i, 0)),
            pl.BlockSpec((_R, _D), lambda i: (i, 0)),
            pl.BlockSpec((_D, _D), lambda i: (0, 0)),
            pl.BlockSpec((1, _D), lambda i: (0, 0)),
            pl.BlockSpec((_D, _D), lambda i: (0, 0)),
            pl.BlockSpec((_D, _D), lambda i: (0, 0)),
        ],
        out_specs=[
            pl.BlockSpec((_R, _D), lambda i: (i, 0)),
            pl.BlockSpec((_R, _D), lambda i: (i, 0)),
            pl.BlockSpec((_R, 1), lambda i: (i, 0)),
        ],
        out_shape=[
            jax.ShapeDtypeStruct((_NP, _D), jnp.float32),
            jax.ShapeDtypeStruct((_NP, _D), jnp.float32),
            jax.ShapeDtypeStruct((_NP, 1), jnp.float32),
        ],
    )(aggP, aggP, hist, x, W1_l, b1, W1_r, W2_r)


def _tc_layer2(aggP, cnt, h1r, W2_l, b2, W_out, b_out):
    def body(a0, a1, c_r, h1r_r, wl, br, wo, bo, y_o):
        mean = (a0[...] + a1[...]) / c_r[...]
        t = lax.dot_general(mean, wl[...], _CONTRACT_T,
                            preferred_element_type=jnp.float32)
        h2 = jnp.maximum(t + br[...] + h1r_r[...], 0.0)
        logit = jnp.sum(h2 * wo[...], axis=1, keepdims=True) + bo[0, 0]
        y_o[...] = jax.nn.sigmoid(logit)

    grid = (_NB,)
    return pl.pallas_call(
        body,
        grid=grid,
        in_specs=[
            pl.BlockSpec((_R, _D), lambda i: (i, 0)),
            pl.BlockSpec((_R, _D), lambda i: (i + _NB, 0)),
            pl.BlockSpec((_R, 1), lambda i: (i, 0)),
            pl.BlockSpec((_R, _D), lambda i: (i, 0)),
            pl.BlockSpec((_D, _D), lambda i: (0, 0)),
            pl.BlockSpec((1, _D), lambda i: (0, 0)),
            pl.BlockSpec((1, _D), lambda i: (0, 0)),
            pl.BlockSpec((1, 1), lambda i: (0, 0)),
        ],
        out_specs=[pl.BlockSpec((_R, 1), lambda i: (i, 0))],
        out_shape=[jax.ShapeDtypeStruct((_NP, 1), jnp.float32)],
    )(aggP, aggP, cnt, h1r, W2_l, b2, W_out, b_out)


def kernel(x, W1_l, b1_l, W1_r, W2_l, b2_l, W2_r, W_out, b_out, edge_index):
    npad = _EPAD - _E
    pad_src = (jnp.arange(npad, dtype=jnp.int32) * 997) % _N
    pad_dst = _N + (jnp.arange(npad, dtype=jnp.int32) % (_NP - _N))
    src = jnp.concatenate([edge_index[0], pad_src]).reshape(_NCHUNK, _CH)
    dst = jnp.concatenate([edge_index[1], pad_dst]).reshape(_NCHUNK, _CH)
    zrow = jnp.zeros((_NP, _D), jnp.float32)

    agg1, hist = _AGG_CNT(src, dst, x, zrow)
    hist = hist.reshape(_NW, _NP).T
    h1, h1r, cnt = _tc_layer1(agg1, hist, x, W1_l, b1_l.reshape(1, _D),
                              W1_r, W2_r)
    agg2 = _AGG(src, dst, h1, zrow)
    if isinstance(agg2, (tuple, list)):
        agg2 = agg2[0]
    y = _tc_layer2(agg2, cnt, h1r, W2_l, b2_l.reshape(1, _D),
                   W_out, b_out.reshape(1, 1))
    if isinstance(y, (tuple, list)):
        y = y[0]
    return y.reshape(_NP)[:_N]


# speedup vs baseline: 1.1048x; 1.1048x over previous
import jax
import jax.numpy as jnp
from jax import lax
from jax.experimental import pallas as pl
from jax.experimental.pallas import tpu as pltpu
from jax.experimental.pallas import tpu_sc as plsc

_N = 10000
_E = 320000
_D = 128
_NC = 2
_NS = 16
_NW = _NC * _NS
_CH = 128
_NCHUNK = 2560
_EPAD = _NCHUNK * _CH
_EPW = _NCHUNK // _NW
_B = 8
_NBLK = _EPW // _B
_NBUF = 2
_NP = 10240
_RPT = _NP // _NS
_R = 640
_NB = _NP // _R


def _build_agg(with_cnt):
    mesh = plsc.VectorSubcoreMesh(core_axis_name="c", subcore_axis_name="s")
    out_type = [jax.ShapeDtypeStruct((_NC * _NP, _D), jnp.float32)]
    scratch = [
        pltpu.VMEM((2, _B, _CH), jnp.int32),
        pltpu.VMEM((2, _B, _CH), jnp.int32),
        pltpu.VMEM((_NBUF, _CH, _D), jnp.float32),
        pltpu.VMEM_SHARED((_NP, _D), jnp.float32),
    ] + [pltpu.SemaphoreType.DMA] * (_NBUF + 1)
    if with_cnt:
        out_type.append(jax.ShapeDtypeStruct((_NW * _NP,), jnp.float32))
        scratch.append(pltpu.VMEM((_NP,), jnp.float32))

    def body(*refs):
        if with_cnt:
            (src_h, dst_h, tab_h, zrow_h,
             agg_h, hist_h,
             src_a, dst_a, rows, acc, *sems_all, hist) = refs
        else:
            (src_h, dst_h, tab_h, zrow_h,
             agg_h,
             src_a, dst_a, rows, acc, *sems_all) = refs
        gsems = sems_all[:_NBUF]
        sem_i = sems_all[_NBUF]
        c = lax.axis_index("c")
        s = lax.axis_index("s")
        wid = s * _NC + c
        base = s * _RPT
        brow0 = wid * _EPW

        def idx_block(kb, slot, do):
            r = brow0 + kb * _B
            a = pltpu.make_async_copy(src_h.at[pl.ds(r, _B)], src_a.at[slot],
                                      sem_i)
            b = pltpu.make_async_copy(dst_h.at[pl.ds(r, _B)], dst_a.at[slot],
                                      sem_i)
            if do == "start":
                a.start()
                b.start()
            else:
                a.wait()
                b.wait()

        pltpu.sync_copy(src_h.at[pl.ds(brow0, _B)], src_a.at[0])
        pltpu.sync_copy(dst_h.at[pl.ds(brow0, _B)], dst_a.at[0])
        pltpu.sync_copy(zrow_h.at[pl.ds(base, _RPT)], acc.at[pl.ds(base, _RPT)])
        if with_cnt:
            def zstep(i, carry):
                hist[pl.ds(i * 16, 16)] = jnp.zeros((16,), jnp.float32)
                return carry
            lax.fori_loop(0, _NP // 16, zstep, 0)
        plsc.subcore_barrier()

        pltpu.async_copy(tab_h.at[src_a.at[0, 0]], rows.at[0], gsems[0])
        pltpu.async_copy(tab_h.at[src_a.at[0, 1]], rows.at[1], gsems[1])
        idx_block(1, 1, "start")

        def step(kb, carry):
            cur = lax.rem(kb, 2)
            nxt = 1 - cur

            @pl.when(kb + 1 < _NBLK)
            def _():
                idx_block(kb + 1, nxt, "wait")

            for j in range(_B):
                b = j % _NBUF
                pltpu.make_async_copy(tab_h.at[src_a.at[cur, j]],
                                      rows.at[b], gsems[b]).wait()
                pltpu.sync_copy(rows.at[b], acc.at[dst_a.at[cur, j]],
                                add=True)
                if j < _B - 2:
                    pltpu.async_copy(tab_h.at[src_a.at[cur, j + 2]],
                                     rows.at[b], gsems[b])
                else:
                    @pl.when(kb + 1 < _NBLK)
                    def _():
                        pltpu.async_copy(tab_h.at[src_a.at[nxt, j + 2 - _B]],
                                         rows.at[b], gsems[b])
                if with_cnt:
                    for i in range(_CH // 16):
                        v = dst_a[cur, j, pl.ds(i * 16, 16)]
                        plsc.addupdate_scatter(
                            hist, [v], jnp.ones((16,), jnp.float32))

            @pl.when(kb + 2 < _NBLK)
            def _():
                idx_block(kb + 2, cur, "start")
            return carry

        lax.fori_loop(0, _NBLK, step, 0)
        plsc.subcore_barrier()
        orow = c * _NP + base
        pltpu.sync_copy(acc.at[pl.ds(base, _RPT)], agg_h.at[pl.ds(orow, _RPT)])
        if with_cnt:
            pltpu.sync_copy(hist, hist_h.at[pl.ds(wid * _NP, _NP)])

    return pl.kernel(body, out_type=tuple(out_type), mesh=mesh,
                     scratch_types=tuple(scratch),
                     compiler_params=pltpu.CompilerParams(
                         needs_layout_passes=not with_cnt))


_AGG_CNT = _build_agg(True)
_AGG = _build_agg(False)

_CONTRACT_T = (((1,), (1,)), ((), ()))


def _tc_layer1(aggP, hist, x, W1_l, b1, W1_r, W2_r):
    def body(a0, a1, h_r, x_r, wl, br, wr, w2r, h1_o, h1r_o, cnt_o):
        cnt = jnp.maximum(jnp.sum(h_r[...], axis=1, keepdims=True), 1.0)
        mean = (a0[...] + a1[...]) / cnt
        t = lax.dot_general(mean, wl[...], _CONTRACT_T,
                            preferred_element_type=jnp.float32)
        u = lax.dot_general(x_r[...], wr[...], _CONTRACT_T,
                            preferred_element_type=jnp.float32)
        h1 = jnp.maximum(t + br[...] + u, 0.0)
        h1_o[...] = h1
        h1r_o[...] = lax.dot_general(h1, w2r[...], _CONTRACT_T,
                                     preferred_element_type=jnp.float32)
        cnt_o[...] = cnt

    grid = (_NB,)
    return pl.pallas_call(
        body,
        grid=grid,
        in_specs=[
            pl.BlockSpec((_R, _D), lambda i: (i, 0)),
            pl.BlockSpec((_R, _D), lambda i: (i + _NB, 0)),
            pl.BlockSpec((_R, _NW), lambda i: (i, 0)),
            pl.BlockSpec((_R, _D), lambda i: (i, 0)),
            pl.BlockSpec((_D, _D), lambda i: (0, 0)),
            pl.BlockSpec((1, _D), lambda i: (0, 0)),
            pl.BlockSpec((_D, _D), lambda i: (0, 0)),
            pl.BlockSpec((_D, _D), lambda i: (0, 0)),
        ],
        out_specs=[
            pl.BlockSpec((_R, _D), lambda i: (i, 0)),
            pl.BlockSpec((_R, _D), lambda i: (i, 0)),
            pl.BlockSpec((_R, 1), lambda i: (i, 0)),
        ],
        out_shape=[
            jax.ShapeDtypeStruct((_NP, _D), jnp.float32),
            jax.ShapeDtypeStruct((_NP, _D), jnp.float32),
            jax.ShapeDtypeStruct((_NP, 1), jnp.float32),
        ],
    )(aggP, aggP, hist, x, W1_l, b1, W1_r, W2_r)


def _tc_layer2(aggP, cnt, h1r, W2_l, b2, W_out, b_out):
    def body(a0, a1, c_r, h1r_r, wl, br, wo, bo, y_o):
        mean = (a0[...] + a1[...]) / c_r[...]
        t = lax.dot_general(mean, wl[...], _CONTRACT_T,
                            preferred_element_type=jnp.float32)
        h2 = jnp.maximum(t + br[...] + h1r_r[...], 0.0)
        logit = jnp.sum(h2 * wo[...], axis=1, keepdims=True) + bo[0, 0]
        y_o[...] = jax.nn.sigmoid(logit)

    grid = (_NB,)
    return pl.pallas_call(
        body,
        grid=grid,
        in_specs=[
            pl.BlockSpec((_R, _D), lambda i: (i, 0)),
            pl.BlockSpec((_R, _D), lambda i: (i + _NB, 0)),
            pl.BlockSpec((_R, 1), lambda i: (i, 0)),
            pl.BlockSpec((_R, _D), lambda i: (i, 0)),
            pl.BlockSpec((_D, _D), lambda i: (0, 0)),
            pl.BlockSpec((1, _D), lambda i: (0, 0)),
            pl.BlockSpec((1, _D), lambda i: (0, 0)),
            pl.BlockSpec((1, 1), lambda i: (0, 0)),
        ],
        out_specs=[pl.BlockSpec((_R, 1), lambda i: (i, 0))],
        out_shape=[jax.ShapeDtypeStruct((_NP, 1), jnp.float32)],
    )(aggP, aggP, cnt, h1r, W2_l, b2, W_out, b_out)


def kernel(x, W1_l, b1_l, W1_r, W2_l, b2_l, W2_r, W_out, b_out, edge_index):
    npad = _EPAD - _E
    pad_src = (jnp.arange(npad, dtype=jnp.int32) * 997) % _N
    pad_dst = _N + (jnp.arange(npad, dtype=jnp.int32) % (_NP - _N))
    src = jnp.concatenate([edge_index[0], pad_src]).reshape(_NCHUNK, _CH)
    dst = jnp.concatenate([edge_index[1], pad_dst]).reshape(_NCHUNK, _CH)
    zrow = jnp.zeros((_NP, _D), jnp.float32)

    agg1, hist = _AGG_CNT(src, dst, x, zrow)
    hist = hist.reshape(_NW, _NP).T
    h1, h1r, cnt = _tc_layer1(agg1, hist, x, W1_l, b1_l.reshape(1, _D),
                              W1_r, W2_r)
    agg2 = _AGG(src, dst, h1, zrow)
    if isinstance(agg2, (tuple, list)):
        agg2 = agg2[0]
    y = _tc_layer2(agg2, cnt, h1r, W2_l, b2_l.reshape(1, _D),
                   W_out, b_out.reshape(1, 1))
    if isinstance(y, (tuple, list)):
        y = y[0]
    return y.reshape(_NP)[:_N]
